# baseline (speedup 1.0000x reference)
import jax
import jax.numpy as jnp
from jax.experimental import pallas as pl

NUM_CATEGORIES = 1000


def _onehot_body(inp_ref, out_ref):
    f = pl.program_id(0)
    v = inp_ref[pl.ds(f, 1), :]
    iota = jax.lax.broadcasted_iota(
        jnp.int32, (1, NUM_CATEGORIES, v.shape[1]), 1
    )
    out_ref[...] = (iota == v[:, None, :]).astype(jnp.float32)


def _half(vt):
    nfeat, batch = vt.shape
    return pl.pallas_call(
        _onehot_body,
        grid=(nfeat,),
        in_specs=[pl.BlockSpec((nfeat, batch), lambda f: (0, 0))],
        out_specs=pl.BlockSpec((1, NUM_CATEGORIES, batch), lambda f: (f, 0, 0)),
        out_shape=jax.ShapeDtypeStruct((nfeat, NUM_CATEGORIES, batch), jnp.float32),
    )(vt)


def kernel(inputs):
    batch, nfeat = inputs.shape
    vt = inputs.astype(jnp.int32).T
    a = _half(vt[:13])
    b = _half(vt[13:])
    out_t = jnp.concatenate([a, b], axis=0)
    return jnp.transpose(out_t, (2, 0, 1))


# speedup vs baseline: 1.5008x; 1.5008x over previous
import dataclasses

import jax
import jax.numpy as jnp
from jax import lax
from jax.experimental import pallas as pl
from jax.experimental.pallas import tpu as pltpu
from jax.experimental.pallas import tpu_sc as plsc

NUM_CATEGORIES = 1000
ROWS_PER_CHUNK = 40
N_SUBCORES = 32


def _sc_onehot(vt):
    nfeat, batch = vt.shape
    n_rows = nfeat * NUM_CATEGORIES
    n_chunks = n_rows // ROWS_PER_CHUNK
    chunks_per_feat = NUM_CATEGORIES // ROWS_PER_CHUNK
    n_iters = n_chunks // (2 * N_SUBCORES) + 2

    mesh = plsc.VectorSubcoreMesh(core_axis_name="c", subcore_axis_name="s")

    cp = pltpu.CompilerParams()
    if "needs_layout_passes" in pltpu.CompilerParams.__dataclass_fields__:
        cp = dataclasses.replace(cp, needs_layout_passes=False)

    @pl.kernel(
        compiler_params=cp,
        out_type=jax.ShapeDtypeStruct((nfeat, NUM_CATEGORIES, batch), jnp.float32),
        mesh=mesh,
        scratch_types=[
            pltpu.VMEM((nfeat, batch), jnp.int32),
            pltpu.VMEM((ROWS_PER_CHUNK, batch), jnp.float32),
            pltpu.VMEM((ROWS_PER_CHUNK, batch), jnp.float32),
            pltpu.SemaphoreType.DMA,
            pltpu.SemaphoreType.DMA,
        ],
    )
    def body(vt_ref, out_ref, vbuf, buf0, buf1, sem0, sem1):
        u = lax.axis_index("c") * 16 + lax.axis_index("s")

        pltpu.sync_copy(vt_ref, vbuf)

        @pl.loop(0, ROWS_PER_CHUNK)
        def _(r):
            @pl.loop(0, batch // 16)
            def _(j):
                z = jnp.zeros((16,), jnp.float32)
                buf0[r, pl.ds(j * 16, 16)] = z
                buf1[r, pl.ds(j * 16, 16)] = z

        def scan_scatter(buf, k, value):
            f = k // chunks_per_feat
            c0 = (k % chunks_per_feat) * ROWS_PER_CHUNK
            val = jnp.full((16,), value, jnp.float32)

            @pl.loop(0, batch // 16)
            def _(j):
                v = vbuf[f, pl.ds(j * 16, 16)]
                rel = v - c0
                mask = (rel >= 0) & (rel < ROWS_PER_CHUNK)
                relc = jnp.clip(rel, 0, ROWS_PER_CHUNK - 1)
                b = lax.iota(jnp.int32, 16) + j * 16
                plsc.store_scatter(buf, [relc, b], val, mask=mask)

        def dma(buf, sem, k):
            f = k // chunks_per_feat
            c0 = (k % chunks_per_feat) * ROWS_PER_CHUNK
            return pltpu.make_async_copy(
                buf, out_ref.at[f, pl.ds(c0, ROWS_PER_CHUNK)], sem
            )

        @pl.loop(0, n_iters)
        def _(t):
            k0 = u + 2 * N_SUBCORES * t
            k0p = k0 - 2 * N_SUBCORES

            @pl.when((t >= 1) & (k0p < n_chunks))
            def _():
                dma(buf0, sem0, k0p).wait()
                scan_scatter(buf0, k0p, 0.0)

            @pl.when(k0 < n_chunks)
            def _():
                scan_scatter(buf0, k0, 1.0)
                dma(buf0, sem0, k0).start()

            k1 = k0 + N_SUBCORES
            k1p = k1 - 2 * N_SUBCORES

            @pl.when((t >= 1) & (k1p < n_chunks))
            def _():
                dma(buf1, sem1, k1p).wait()
                scan_scatter(buf1, k1p, 0.0)

            @pl.when(k1 < n_chunks)
            def _():
                scan_scatter(buf1, k1, 1.0)
                dma(buf1, sem1, k1).start()

    return body(vt)


def kernel(inputs):
    batch, nfeat = inputs.shape
    vt = inputs.astype(jnp.int32).T
    out_t = _sc_onehot(vt)
    return jnp.transpose(out_t, (2, 0, 1))


# speedup vs baseline: 3.1186x; 2.0780x over previous
import jax
import jax.numpy as jnp
from jax.experimental import pallas as pl

NUM_CATEGORIES = 1000


def _onehot_body(inp_ref, out_ref):
    f = pl.program_id(0)
    v = inp_ref[pl.ds(f, 1), :]
    iota = jax.lax.broadcasted_iota(
        jnp.int32, (1, NUM_CATEGORIES, v.shape[1]), 1
    )
    out_ref[...] = (iota == v[:, None, :]).astype(jnp.float32)


def kernel(inputs):
    batch, nfeat = inputs.shape
    vt = inputs.astype(jnp.int32).T
    out_t = pl.pallas_call(
        _onehot_body,
        grid=(nfeat,),
        in_specs=[pl.BlockSpec((nfeat, batch), lambda f: (0, 0))],
        out_specs=pl.BlockSpec((1, NUM_CATEGORIES, batch), lambda f: (f, 0, 0)),
        out_shape=jax.ShapeDtypeStruct((nfeat, NUM_CATEGORIES, batch), jnp.float32),
    )(vt)
    return jnp.transpose(out_t, (2, 0, 1))
